# baseline (speedup 1.0000x reference)
import functools

import jax
import jax.numpy as jnp
from jax.experimental import pallas as pl
from jax.experimental.pallas import tpu as pltpu
from jax.experimental.pallas import tpu_sc as plsc
from jax import lax

N = 2048
E = 65536
IN_FEATS = 256
N_HIDDEN = 128
N_CLASSES = 64
EPS = 1e-6
EC = 128
NCHUNK = E // EC

_INTERPRET = False

_LNOISE = None


def _lnoise_const():
    global _LNOISE
    if _LNOISE is None:
        with jax.ensure_compile_time_eval():
            u = jax.random.uniform(jax.random.key(42), (N, N),
                                   minval=EPS, maxval=1.0 - EPS,
                                   dtype=jnp.float32)
            _LNOISE = jnp.log(u) - jnp.log1p(-u)
    return _LNOISE


def _pc(body, **kw):
    return pl.pallas_call(body, interpret=_INTERPRET, **kw)


NW = 32
EPW = E // NW
KCH = 128
NCH = EPW // KCH


def _sc_deg_body(src_hbm, dst_hbm, zeros_hbm, ones_hbm,
                 degO_hbm, degI_hbm,
                 idx_v, ones_v, shO, shI, sem):
    c = lax.axis_index("c")
    sid = lax.axis_index("s")
    wid = sid * 2 + c

    @pl.when(sid == 0)
    def _():
        pltpu.sync_copy(zeros_hbm, shO)
        pltpu.sync_copy(zeros_hbm, shI)

    plsc.subcore_barrier()
    pltpu.sync_copy(ones_hbm, ones_v)
    base = wid * EPW

    def step(k, _):
        pltpu.sync_copy(src_hbm.at[pl.ds(base + k * KCH, KCH)], idx_v)
        pltpu.sync_copy(ones_v, shO.at[idx_v], add=True)
        pltpu.sync_copy(dst_hbm.at[pl.ds(base + k * KCH, KCH)], idx_v)
        pltpu.sync_copy(ones_v, shI.at[idx_v], add=True)
        return 0

    lax.fori_loop(0, NCH, step, 0)
    plsc.subcore_barrier()

    @pl.when(sid == 0)
    def _():
        pltpu.sync_copy(shO, degO_hbm.at[c])
        pltpu.sync_copy(shI, degI_hbm.at[c])


def _sc_agg_body(y_hbm, src_hbm, dst_hbm, zeros_hbm,
                 agg_hbm,
                 idx_s, idx_d, rows_v, sh, sem):
    c = lax.axis_index("c")
    sid = lax.axis_index("s")
    wid = sid * 2 + c

    @pl.when(sid == 0)
    def _():
        pltpu.sync_copy(zeros_hbm, sh)

    plsc.subcore_barrier()
    base = wid * EPW

    def step(k, _):
        pltpu.sync_copy(src_hbm.at[pl.ds(base + k * KCH, KCH)], idx_s)
        pltpu.sync_copy(dst_hbm.at[pl.ds(base + k * KCH, KCH)], idx_d)
        pltpu.async_copy(y_hbm.at[idx_s], rows_v, sem).wait()
        pltpu.sync_copy(rows_v, sh.at[idx_d], add=True)
        return 0

    lax.fori_loop(0, NCH, step, 0)
    plsc.subcore_barrier()

    @pl.when(sid == 0)
    def _():
        pltpu.sync_copy(sh, agg_hbm.at[c])


def _sc_deg(src, dst):
    zeros16 = jnp.zeros((N, 128), jnp.float32)
    ones16 = jnp.ones((KCH, 128), jnp.float32)
    mesh = plsc.VectorSubcoreMesh(core_axis_name="c", subcore_axis_name="s")
    f = pl.kernel(
        _sc_deg_body, mesh=mesh,
        out_type=[jax.ShapeDtypeStruct((2, N, 128), jnp.float32),
                  jax.ShapeDtypeStruct((2, N, 128), jnp.float32)],
        scratch_types=[pltpu.VMEM((KCH,), jnp.int32),
                       pltpu.VMEM((KCH, 128), jnp.float32),
                       pltpu.VMEM_SHARED((N, 128), jnp.float32),
                       pltpu.VMEM_SHARED((N, 128), jnp.float32),
                       pltpu.SemaphoreType.DMA])
    return f(src, dst, zeros16, ones16)


def _sc_agg(y, src, dst):
    zeros = jnp.zeros((N, N_HIDDEN), jnp.float32)
    mesh = plsc.VectorSubcoreMesh(core_axis_name="c", subcore_axis_name="s")
    f = pl.kernel(
        _sc_agg_body, mesh=mesh,
        out_type=jax.ShapeDtypeStruct((2, N, N_HIDDEN), jnp.float32),
        scratch_types=[pltpu.VMEM((KCH,), jnp.int32),
                       pltpu.VMEM((KCH,), jnp.int32),
                       pltpu.VMEM((KCH, N_HIDDEN), jnp.float32),
                       pltpu.VMEM_SHARED((N, N_HIDDEN), jnp.float32),
                       pltpu.SemaphoreType.DMA])
    return f(y, src, dst, zeros)


def _y_body(dOp_ref, x_ref, w_ref, y_ref):
    degO = dOp_ref[0:N, 0:1] + dOp_ref[N:2 * N, 0:1]
    nS = lax.rsqrt(jnp.clip(degO, 1.0, None))
    y_ref[...] = jnp.dot(x_ref[...] * nS, w_ref[...],
                         preferred_element_type=jnp.float32)


def _make_comb_body(relu):
    def body(aggP_ref, dIp_ref, b_ref, o_ref):
        degI = dIp_ref[0:N, 0:1] + dIp_ref[N:2 * N, 0:1]
        nD = lax.rsqrt(jnp.clip(degI, 1.0, None))
        a = aggP_ref[0:N, :] + aggP_ref[N:2 * N, :]
        o = a * nD + b_ref[...]
        o_ref[...] = jnp.maximum(o, 0.0) if relu else o
    return body


def _dec_body(h_ref, hall_ref, z_ref, zmax_ref):
    i = pl.program_id(0)
    zb = jax.lax.dot_general(h_ref[...], hall_ref[...], (((1,), (1,)), ((), ())),
                             preferred_element_type=jnp.float32)
    z_ref[...] = zb
    m = jnp.max(zb)

    @pl.when(i == 0)
    def _():
        zmax_ref[...] = jnp.full((1, 1), m, jnp.float32)

    @pl.when(i > 0)
    def _():
        zmax_ref[...] = jnp.maximum(zmax_ref[...], m)


def _mask_body(zij_ref, zji_ref, lij_ref, lji_ref, zmax_ref, m_ref):
    i = pl.program_id(0)
    j = pl.program_id(1)
    gi = i * 128 + jax.lax.broadcasted_iota(jnp.int32, (128, 128), 0)
    gj = j * 128 + jax.lax.broadcasted_iota(jnp.int32, (128, 128), 1)
    upper = gi < gj
    zv = jnp.where(upper, zij_ref[...], zji_ref[...].T)
    lv = jnp.where(upper, lij_ref[...], lji_ref[...].T)
    ep = jax.nn.sigmoid(zv)
    epmax = jax.nn.sigmoid(zmax_ref[...])
    mm = ep / epmax
    pc = jnp.clip(mm, EPS, 1.0 - EPS)
    logits = jnp.log(pc) - jnp.log1p(-pc)
    hard = (logits + lv) > 0.0
    mv = jnp.logical_or(gi == gj, jnp.logical_and(gi != gj, hard))
    m_ref[...] = jnp.where(mv, 0.0, -1e30).astype(jnp.bfloat16)


def _pool_body(h_ref, w_ref, b_ref, o_ref):
    o_ref[...] = jnp.maximum(
        jax.lax.dot_general(h_ref[...], w_ref[...], (((1,), (1,)), ((), ())),
                            preferred_element_type=jnp.float32)
        + b_ref[...], 0.0).astype(jnp.bfloat16)


def _make_agg_body(d_in, final):
    def body(m_ref, msgs_ref, w_ref, b1_ref, b2_ref, o_ref):
        def step(c, accs):
            off = pl.multiple_of(c * 128, 128)
            penc = m_ref[:, pl.ds(off, 128)]
            mmc = msgs_ref[pl.ds(off, 128), :]
            accs = list(accs)
            for k in range(128):
                cand = penc[:, k:k + 1] + mmc[k:k + 1, :]
                accs[k & 3] = jnp.maximum(accs[k & 3], cand)
            return tuple(accs)

        acc0 = jnp.full((128, d_in), -1e30, jnp.bfloat16)
        a0, a1, a2, a3 = jax.lax.fori_loop(0, N // 128, step,
                                           (acc0, acc0, acc0, acc0))
        agg = jnp.maximum(jnp.maximum(a0, a1), jnp.maximum(a2, a3))
        out = (jax.lax.dot_general(agg.astype(jnp.float32), w_ref[...],
                                   (((1,), (1,)), ((), ())),
                                   preferred_element_type=jnp.float32)
               + b1_ref[...] + b2_ref[...])
        if final:
            o_ref[...] = out
        else:
            r = jnp.maximum(out, 0.0)
            nrm = jnp.sqrt(jnp.sum(r * r, axis=1, keepdims=True))
            o_ref[...] = r / jnp.maximum(nrm, 1e-12)
    return body


def _sage_layer(mask8, h, pool_W, pool_b, lin_W, lin_b, bias, final):
    d_in = h.shape[1]
    d_out = lin_W.shape[0]
    msgs = _pc(
        _pool_body,
        grid=(N // 128,),
        in_specs=[pl.BlockSpec((128, d_in), lambda i: (i, 0)),
                  pl.BlockSpec((d_in, d_in), lambda i: (0, 0)),
                  pl.BlockSpec((1, d_in), lambda i: (0, 0))],
        out_specs=pl.BlockSpec((128, d_in), lambda i: (i, 0)),
        out_shape=jax.ShapeDtypeStruct((N, d_in), jnp.bfloat16),
    )(h, pool_W, pool_b.reshape(1, -1))
    return _pc(
        _make_agg_body(d_in, final),
        grid=(N // 128,),
        in_specs=[pl.BlockSpec((128, N), lambda i: (i, 0)),
                  pl.BlockSpec((N, d_in), lambda i: (0, 0)),
                  pl.BlockSpec((d_out, d_in), lambda i: (0, 0)),
                  pl.BlockSpec((1, d_out), lambda i: (0, 0)),
                  pl.BlockSpec((1, d_out), lambda i: (0, 0))],
        out_specs=pl.BlockSpec((128, d_out), lambda i: (i, 0)),
        out_shape=jax.ShapeDtypeStruct((N, d_out), jnp.float32),
    )(
        mask8, msgs, lin_W, lin_b.reshape(1, -1), bias.reshape(1, -1))


def kernel(adj, edge_index, inputs, feat_inputs, gc0_W, gc0_b, gc1_W, gc1_b,
           p0_pool_W, p0_pool_b, p0_lin_W, p0_lin_b, p0_bias,
           p1_pool_W, p1_pool_b, p1_lin_W, p1_lin_b, p1_bias,
           p2_pool_W, p2_pool_b, p2_lin_W, p2_lin_b, p2_bias):
    src = edge_index[0]
    dst = edge_index[1]
    lnoise = _lnoise_const()

    f32 = functools.partial(jax.ShapeDtypeStruct, dtype=jnp.float32)

    degOp, degIp = _sc_deg(src, dst)
    degOp = degOp.reshape(2 * N, 128)
    degIp = degIp.reshape(2 * N, 128)

    def yk(x, w):
        d_in = x.shape[1]
        return _pc(_y_body, out_shape=f32((N, N_HIDDEN)))(degOp, x, w)

    def comb(aggP, b, relu):
        return _pc(_make_comb_body(relu), out_shape=f32((N, N_HIDDEN)))(
            aggP.reshape(2 * N, N_HIDDEN), degIp, b.reshape(1, -1))

    y0 = yk(inputs, gc0_W)
    h1 = comb(_sc_agg(y0, src, dst), gc0_b, True)
    y1 = yk(h1, gc1_W)
    henc = comb(_sc_agg(y1, src, dst), gc1_b, False)

    z, zmax = _pc(
        _dec_body,
        grid=(N // 128,),
        in_specs=[pl.BlockSpec((128, N_HIDDEN), lambda i: (i, 0)),
                  pl.BlockSpec((N, N_HIDDEN), lambda i: (0, 0))],
        out_specs=[pl.BlockSpec((128, N), lambda i: (i, 0)),
                   pl.BlockSpec((1, 1), lambda i: (0, 0))],
        out_shape=[f32((N, N)), f32((1, 1))],
    )(henc, henc)

    mask8 = _pc(
        _mask_body,
        grid=(N // 128, N // 128),
        in_specs=[pl.BlockSpec((128, 128), lambda i, j: (i, j)),
                  pl.BlockSpec((128, 128), lambda i, j: (j, i)),
                  pl.BlockSpec((128, 128), lambda i, j: (i, j)),
                  pl.BlockSpec((128, 128), lambda i, j: (j, i)),
                  pl.BlockSpec((1, 1), lambda i, j: (0, 0))],
        out_specs=pl.BlockSpec((128, 128), lambda i, j: (i, j)),
        out_shape=jax.ShapeDtypeStruct((N, N), jnp.bfloat16),
    )(z, z, lnoise, lnoise, zmax)

    h2 = _sage_layer(mask8, feat_inputs, p0_pool_W, p0_pool_b,
                     p0_lin_W, p0_lin_b, p0_bias, final=False)
    h2 = _sage_layer(mask8, h2, p1_pool_W, p1_pool_b,
                     p1_lin_W, p1_lin_b, p1_bias, final=False)
    h2 = _sage_layer(mask8, h2, p2_pool_W, p2_pool_b,
                     p2_lin_W, p2_lin_b, p2_bias, final=True)
    return h2


# speedup vs baseline: 1.0355x; 1.0355x over previous
import functools

import jax
import jax.numpy as jnp
from jax.experimental import pallas as pl
from jax.experimental.pallas import tpu as pltpu
from jax.experimental.pallas import tpu_sc as plsc
from jax import lax

N = 2048
E = 65536
IN_FEATS = 256
N_HIDDEN = 128
N_CLASSES = 64
EPS = 1e-6
_LNOISE = None


def _lnoise_const():
    global _LNOISE
    if _LNOISE is None:
        with jax.ensure_compile_time_eval():
            u = jax.random.uniform(jax.random.key(42), (N, N),
                                   minval=EPS, maxval=1.0 - EPS,
                                   dtype=jnp.float32)
            _LNOISE = jnp.log(u) - jnp.log1p(-u)
    return _LNOISE


def _pc(body, **kw):
    return pl.pallas_call(body, **kw)


NW = 32
EPW = E // NW
KCH = 128
NCH = EPW // KCH


def _sc_deg_body(src_hbm, dst_hbm, zeros_hbm, ones_hbm,
                 degO_hbm, degI_hbm,
                 idx_v, ones_v, shO, shI, sem):
    c = lax.axis_index("c")
    sid = lax.axis_index("s")
    wid = sid * 2 + c

    @pl.when(sid == 0)
    def _():
        pltpu.sync_copy(zeros_hbm, shO)
        pltpu.sync_copy(zeros_hbm, shI)

    plsc.subcore_barrier()
    pltpu.sync_copy(ones_hbm, ones_v)
    base = wid * EPW

    def step(k, _):
        pltpu.sync_copy(src_hbm.at[pl.ds(base + k * KCH, KCH)], idx_v)
        pltpu.sync_copy(ones_v, shO.at[idx_v], add=True)
        pltpu.sync_copy(dst_hbm.at[pl.ds(base + k * KCH, KCH)], idx_v)
        pltpu.sync_copy(ones_v, shI.at[idx_v], add=True)
        return 0

    lax.fori_loop(0, NCH, step, 0)
    plsc.subcore_barrier()

    @pl.when(sid == 0)
    def _():
        pltpu.sync_copy(shO, degO_hbm.at[c])
        pltpu.sync_copy(shI, degI_hbm.at[c])


def _sc_agg_body(y_hbm, src_hbm, dst_hbm, zeros_hbm,
                 agg_hbm,
                 idx_s, idx_d, rows_v, sh, sem):
    c = lax.axis_index("c")
    sid = lax.axis_index("s")
    wid = sid * 2 + c

    @pl.when(sid == 0)
    def _():
        pltpu.sync_copy(zeros_hbm, sh)

    plsc.subcore_barrier()
    base = wid * EPW

    def step(k, _):
        pltpu.sync_copy(src_hbm.at[pl.ds(base + k * KCH, KCH)], idx_s)
        pltpu.sync_copy(dst_hbm.at[pl.ds(base + k * KCH, KCH)], idx_d)
        pltpu.async_copy(y_hbm.at[idx_s], rows_v, sem).wait()
        pltpu.sync_copy(rows_v, sh.at[idx_d], add=True)
        return 0

    lax.fori_loop(0, NCH, step, 0)
    plsc.subcore_barrier()

    @pl.when(sid == 0)
    def _():
        pltpu.sync_copy(sh, agg_hbm.at[c])


def _sc_deg(src, dst):
    zeros16 = jnp.zeros((N, 128), jnp.float32)
    ones16 = jnp.ones((KCH, 128), jnp.float32)
    mesh = plsc.VectorSubcoreMesh(core_axis_name="c", subcore_axis_name="s")
    f = pl.kernel(
        _sc_deg_body, mesh=mesh,
        out_type=[jax.ShapeDtypeStruct((2, N, 128), jnp.float32),
                  jax.ShapeDtypeStruct((2, N, 128), jnp.float32)],
        scratch_types=[pltpu.VMEM((KCH,), jnp.int32),
                       pltpu.VMEM((KCH, 128), jnp.float32),
                       pltpu.VMEM_SHARED((N, 128), jnp.float32),
                       pltpu.VMEM_SHARED((N, 128), jnp.float32),
                       pltpu.SemaphoreType.DMA])
    return f(src, dst, zeros16, ones16)


def _sc_agg(y, src, dst):
    zeros = jnp.zeros((N, N_HIDDEN), jnp.float32)
    mesh = plsc.VectorSubcoreMesh(core_axis_name="c", subcore_axis_name="s")
    f = pl.kernel(
        _sc_agg_body, mesh=mesh,
        out_type=jax.ShapeDtypeStruct((2, N, N_HIDDEN), jnp.float32),
        scratch_types=[pltpu.VMEM((KCH,), jnp.int32),
                       pltpu.VMEM((KCH,), jnp.int32),
                       pltpu.VMEM((KCH, N_HIDDEN), jnp.float32),
                       pltpu.VMEM_SHARED((N, N_HIDDEN), jnp.float32),
                       pltpu.SemaphoreType.DMA])
    return f(y, src, dst, zeros)


def _y_body(dOp_ref, x_ref, w_ref, y_ref):
    degO = dOp_ref[0:N, 0:1] + dOp_ref[N:2 * N, 0:1]
    nS = lax.rsqrt(jnp.clip(degO, 1.0, None))
    y_ref[...] = jnp.dot(x_ref[...] * nS, w_ref[...],
                         preferred_element_type=jnp.float32)


def _make_comb_body(relu):
    def body(aggP_ref, dIp_ref, b_ref, o_ref):
        degI = dIp_ref[0:N, 0:1] + dIp_ref[N:2 * N, 0:1]
        nD = lax.rsqrt(jnp.clip(degI, 1.0, None))
        a = aggP_ref[0:N, :] + aggP_ref[N:2 * N, :]
        o = a * nD + b_ref[...]
        o_ref[...] = jnp.maximum(o, 0.0) if relu else o
    return body


def _dec_body(h_ref, hall_ref, z_ref, zmax_ref):
    i = pl.program_id(0)
    zb = jax.lax.dot_general(h_ref[...], hall_ref[...], (((1,), (1,)), ((), ())),
                             preferred_element_type=jnp.float32)
    z_ref[...] = zb
    m = jnp.max(zb)

    @pl.when(i == 0)
    def _():
        zmax_ref[...] = jnp.full((1, 1), m, jnp.float32)

    @pl.when(i > 0)
    def _():
        zmax_ref[...] = jnp.maximum(zmax_ref[...], m)


def _mask_body(zij_ref, zji_ref, lij_ref, lji_ref, zmax_ref, m_ref):
    i = pl.program_id(0)
    j = pl.program_id(1)
    gi = i * 128 + jax.lax.broadcasted_iota(jnp.int32, (128, 128), 0)
    gj = j * 128 + jax.lax.broadcasted_iota(jnp.int32, (128, 128), 1)
    upper = gi < gj
    zv = jnp.where(upper, zij_ref[...], zji_ref[...].T)
    lv = jnp.where(upper, lij_ref[...], lji_ref[...].T)
    ep = jax.nn.sigmoid(zv)
    epmax = jax.nn.sigmoid(zmax_ref[...])
    mm = ep / epmax
    pc = jnp.clip(mm, EPS, 1.0 - EPS)
    logits = jnp.log(pc) - jnp.log1p(-pc)
    hard = (logits + lv) > 0.0
    mv = jnp.logical_or(gi == gj, jnp.logical_and(gi != gj, hard))
    m_ref[...] = jnp.where(mv, 0.0, -1e30).astype(jnp.bfloat16)


def _pool_body(h_ref, w_ref, b_ref, o_ref):
    o_ref[...] = jnp.maximum(
        jax.lax.dot_general(h_ref[...], w_ref[...], (((1,), (1,)), ((), ())),
                            preferred_element_type=jnp.float32)
        + b_ref[...], 0.0).astype(jnp.bfloat16)


def _make_agg_body(d_in, final):
    def body(m_ref, msgs_ref, w_ref, b1_ref, b2_ref, o_ref):
        def step(c, acc):
            off = pl.multiple_of(c * 128, 128)
            penc = m_ref[:, pl.ds(off, 128)]
            mmc = msgs_ref[pl.ds(off, 128), :]
            for k in range(128):
                cand = penc[:, k:k + 1] + mmc[k:k + 1, :]
                acc = jnp.maximum(acc, cand)
            return acc

        acc0 = jnp.full((128, d_in), -1e30, jnp.bfloat16)
        agg = jax.lax.fori_loop(0, N // 128, step, acc0)
        out = (jax.lax.dot_general(agg.astype(jnp.float32), w_ref[...],
                                   (((1,), (1,)), ((), ())),
                                   preferred_element_type=jnp.float32)
               + b1_ref[...] + b2_ref[...])
        if final:
            o_ref[...] = out
        else:
            r = jnp.maximum(out, 0.0)
            nrm = jnp.sqrt(jnp.sum(r * r, axis=1, keepdims=True))
            o_ref[...] = r / jnp.maximum(nrm, 1e-12)
    return body


def _sage_layer(mask8, h, pool_W, pool_b, lin_W, lin_b, bias, final):
    d_in = h.shape[1]
    d_out = lin_W.shape[0]
    msgs = _pc(
        _pool_body,
        grid=(N // 128,),
        in_specs=[pl.BlockSpec((128, d_in), lambda i: (i, 0)),
                  pl.BlockSpec((d_in, d_in), lambda i: (0, 0)),
                  pl.BlockSpec((1, d_in), lambda i: (0, 0))],
        out_specs=pl.BlockSpec((128, d_in), lambda i: (i, 0)),
        out_shape=jax.ShapeDtypeStruct((N, d_in), jnp.bfloat16),
    )(h, pool_W, pool_b.reshape(1, -1))
    return _pc(
        _make_agg_body(d_in, final),
        grid=(N // 128,),
        in_specs=[pl.BlockSpec((128, N), lambda i: (i, 0)),
                  pl.BlockSpec((N, d_in), lambda i: (0, 0)),
                  pl.BlockSpec((d_out, d_in), lambda i: (0, 0)),
                  pl.BlockSpec((1, d_out), lambda i: (0, 0)),
                  pl.BlockSpec((1, d_out), lambda i: (0, 0))],
        out_specs=pl.BlockSpec((128, d_out), lambda i: (i, 0)),
        out_shape=jax.ShapeDtypeStruct((N, d_out), jnp.float32),
    )(
        mask8, msgs, lin_W, lin_b.reshape(1, -1), bias.reshape(1, -1))


def kernel(adj, edge_index, inputs, feat_inputs, gc0_W, gc0_b, gc1_W, gc1_b,
           p0_pool_W, p0_pool_b, p0_lin_W, p0_lin_b, p0_bias,
           p1_pool_W, p1_pool_b, p1_lin_W, p1_lin_b, p1_bias,
           p2_pool_W, p2_pool_b, p2_lin_W, p2_lin_b, p2_bias):
    src = edge_index[0]
    dst = edge_index[1]
    lnoise = _lnoise_const()

    f32 = functools.partial(jax.ShapeDtypeStruct, dtype=jnp.float32)

    degOp, degIp = _sc_deg(src, dst)
    degOp = degOp.reshape(2 * N, 128)
    degIp = degIp.reshape(2 * N, 128)

    def yk(x, w):
        return _pc(_y_body, out_shape=f32((N, N_HIDDEN)))(degOp, x, w)

    def comb(aggP, b, relu):
        return _pc(_make_comb_body(relu), out_shape=f32((N, N_HIDDEN)))(
            aggP.reshape(2 * N, N_HIDDEN), degIp, b.reshape(1, -1))

    y0 = yk(inputs, gc0_W)
    h1 = comb(_sc_agg(y0, src, dst), gc0_b, True)
    y1 = yk(h1, gc1_W)
    henc = comb(_sc_agg(y1, src, dst), gc1_b, False)

    z, zmax = _pc(
        _dec_body,
        grid=(N // 128,),
        in_specs=[pl.BlockSpec((128, N_HIDDEN), lambda i: (i, 0)),
                  pl.BlockSpec((N, N_HIDDEN), lambda i: (0, 0))],
        out_specs=[pl.BlockSpec((128, N), lambda i: (i, 0)),
                   pl.BlockSpec((1, 1), lambda i: (0, 0))],
        out_shape=[f32((N, N)), f32((1, 1))],
    )(henc, henc)

    mask8 = _pc(
        _mask_body,
        grid=(N // 128, N // 128),
        in_specs=[pl.BlockSpec((128, 128), lambda i, j: (i, j)),
                  pl.BlockSpec((128, 128), lambda i, j: (j, i)),
                  pl.BlockSpec((128, 128), lambda i, j: (i, j)),
                  pl.BlockSpec((128, 128), lambda i, j: (j, i)),
                  pl.BlockSpec((1, 1), lambda i, j: (0, 0))],
        out_specs=pl.BlockSpec((128, 128), lambda i, j: (i, j)),
        out_shape=jax.ShapeDtypeStruct((N, N), jnp.bfloat16),
    )(z, z, lnoise, lnoise, zmax)

    h2 = _sage_layer(mask8, feat_inputs, p0_pool_W, p0_pool_b,
                     p0_lin_W, p0_lin_b, p0_bias, final=False)
    h2 = _sage_layer(mask8, h2, p1_pool_W, p1_pool_b,
                     p1_lin_W, p1_lin_b, p1_bias, final=False)
    h2 = _sage_layer(mask8, h2, p2_pool_W, p2_pool_b,
                     p2_lin_W, p2_lin_b, p2_bias, final=True)
    return h2
